# baseline (speedup 1.0000x reference)
import functools

import jax
import jax.numpy as jnp
from jax import lax
from jax.experimental import pallas as pl
from jax.experimental.pallas import tpu as pltpu
from jax.experimental.pallas import tpu_sc as plsc

_LANES = 16


def _sc_partial_hist(edges, num_nodes, num_workers):
    num_edges = edges.shape[1]
    lane_tiles = num_edges // 128
    tiles_main = lane_tiles // num_workers
    chunk_main = tiles_main * 128
    chunk_last = num_edges - chunk_main * (num_workers - 1)
    assert chunk_main % _LANES == 0 and chunk_last % _LANES == 0

    mesh = plsc.VectorSubcoreMesh(core_axis_name="c", subcore_axis_name="s")
    num_cores = mesh.num_cores

    @functools.partial(
        pl.kernel,
        out_type=jax.ShapeDtypeStruct((num_workers, num_nodes), jnp.float32),
        mesh=mesh,
        scratch_types=[
            pltpu.VMEM((2, chunk_last), jnp.int32),
            pltpu.VMEM((num_nodes,), jnp.float32),
        ],
        compiler_params=pltpu.CompilerParams(needs_layout_passes=False),
    )
    def hist_kernel(edge_hbm, out_hbm, idx_v, acc_v):
        wid = lax.axis_index("s") * num_cores + lax.axis_index("c")
        base = pl.multiple_of(wid * chunk_main, 128)
        pltpu.sync_copy(edge_hbm.at[:, pl.ds(base, chunk_last)], idx_v)

        zeros = jnp.zeros((_LANES,), jnp.float32)

        def zero_body(i, c):
            acc_v[pl.ds(i * _LANES, _LANES)] = zeros
            return c

        lax.fori_loop(0, num_nodes // _LANES, zero_body, 0, unroll=8)

        ones = jnp.ones((_LANES,), jnp.float32)

        def scat_body(i, c):
            idx = idx_v[0, pl.ds(i * _LANES, _LANES)]
            plsc.addupdate_scatter(acc_v, [idx], ones)
            return c

        lax.fori_loop(0, chunk_main // _LANES, scat_body, 0, unroll=8)

        @pl.when(wid == num_workers - 1)
        def _():
            lax.fori_loop(
                chunk_main // _LANES, chunk_last // _LANES, scat_body, 0, unroll=8
            )

        pltpu.sync_copy(acc_v, out_hbm.at[wid])

    return hist_kernel(edges)


def _tc_finish(x, partials, block_rows):
    num_nodes, d = x.shape
    num_workers = partials.shape[0]
    num_blocks = pl.cdiv(num_nodes, block_rows)

    def copy_body(x_ref, out_ref):
        out_ref[...] = jnp.transpose(x_ref[...], (1, 0))

    staged = pl.pallas_call(
        copy_body,
        grid=(num_blocks,),
        in_specs=[pl.BlockSpec((block_rows, d), lambda i: (i, 0))],
        out_specs=pl.BlockSpec((d, block_rows), lambda i: (0, i)),
        out_shape=jax.ShapeDtypeStruct((d + 1, num_nodes), jnp.float32),
    )(x)

    assert d % 8 == 0
    def col_body(staged_ref, part_ref, out_ref):
        del staged_ref
        deg = jnp.sum(part_ref[...], axis=0, keepdims=True)
        m = jnp.max(deg)
        nrm = jnp.where(m > 0, deg / m, deg)
        out_ref[...] = jnp.broadcast_to(nrm, (8, num_nodes))

    return pl.pallas_call(
        col_body,
        grid=(1,),
        in_specs=[
            pl.BlockSpec(memory_space=pl.ANY),
            pl.BlockSpec((num_workers, num_nodes), lambda i: (0, 0)),
        ],
        out_specs=pl.BlockSpec((8, num_nodes), lambda i: (d // 8, 0)),
        out_shape=jax.ShapeDtypeStruct((d + 1, num_nodes), jnp.float32),
        input_output_aliases={0: 0},
    )(staged, partials)


def kernel(x, edge_index):
    num_nodes = x.shape[0]
    num_workers = 32

    edges = edge_index
    if edges.dtype != jnp.int32:
        edges = edges.astype(jnp.int32)
    partials = _sc_partial_hist(edges, num_nodes, num_workers)
    out_t = _tc_finish(x, partials, block_rows=2560)
    return jnp.transpose(out_t, (1, 0))


# speedup vs baseline: 1.0134x; 1.0134x over previous
import functools

import jax
import jax.numpy as jnp
from jax import lax
from jax.experimental import pallas as pl
from jax.experimental.pallas import tpu as pltpu
from jax.experimental.pallas import tpu_sc as plsc

_LANES = 16


def _sc_partial_hist(edges, num_nodes, num_workers):
    num_edges = edges.shape[1]
    lane_tiles = num_edges // 128
    tiles_main = lane_tiles // num_workers
    chunk_main = tiles_main * 128
    chunk_last = num_edges - chunk_main * (num_workers - 1)
    assert chunk_main % _LANES == 0 and chunk_last % _LANES == 0

    mesh = plsc.VectorSubcoreMesh(core_axis_name="c", subcore_axis_name="s")
    num_cores = mesh.num_cores

    @functools.partial(
        pl.kernel,
        out_type=jax.ShapeDtypeStruct((num_workers, num_nodes), jnp.float32),
        mesh=mesh,
        scratch_types=[
            pltpu.VMEM((2, chunk_last), jnp.int32),
            pltpu.VMEM((num_nodes,), jnp.float32),
        ],
        compiler_params=pltpu.CompilerParams(needs_layout_passes=False),
    )
    def hist_kernel(edge_hbm, out_hbm, idx_v, acc_v):
        wid = lax.axis_index("s") * num_cores + lax.axis_index("c")
        base = pl.multiple_of(wid * chunk_main, 128)
        pltpu.sync_copy(edge_hbm.at[:, pl.ds(base, chunk_last)], idx_v)

        zeros = jnp.zeros((_LANES,), jnp.float32)

        def zero_body(i, c):
            acc_v[pl.ds(i * _LANES, _LANES)] = zeros
            return c

        lax.fori_loop(0, num_nodes // _LANES, zero_body, 0, unroll=8)

        ones = jnp.ones((_LANES,), jnp.float32)

        def scat_body(i, c):
            idx = idx_v[0, pl.ds(i * _LANES, _LANES)]
            plsc.addupdate_scatter(acc_v, [idx], ones)
            return c

        lax.fori_loop(0, chunk_main // _LANES, scat_body, 0, unroll=8)

        @pl.when(wid == num_workers - 1)
        def _():
            lax.fori_loop(
                chunk_main // _LANES, chunk_last // _LANES, scat_body, 0, unroll=8
            )

        pltpu.sync_copy(acc_v, out_hbm.at[wid])

    return hist_kernel(edges)


def _tc_finish(x, partials, block_rows):
    num_nodes, d = x.shape
    num_workers = partials.shape[0]
    num_blocks = pl.cdiv(num_nodes, block_rows)

    def copy_body(x_ref, out_ref):
        eye = jnp.eye(d, dtype=jnp.float32)
        out_ref[...] = lax.dot_general(
            eye, x_ref[...], (((1,), (1,)), ((), ())),
            preferred_element_type=jnp.float32,
        )

    staged = pl.pallas_call(
        copy_body,
        grid=(num_blocks,),
        in_specs=[pl.BlockSpec((block_rows, d), lambda i: (i, 0))],
        out_specs=pl.BlockSpec((d, block_rows), lambda i: (0, i)),
        out_shape=jax.ShapeDtypeStruct((d + 1, num_nodes), jnp.float32),
    )(x)

    assert d % 8 == 0
    def col_body(staged_ref, part_ref, out_ref):
        del staged_ref
        deg = jnp.sum(part_ref[...], axis=0, keepdims=True)
        m = jnp.max(deg)
        nrm = jnp.where(m > 0, deg / m, deg)
        out_ref[...] = jnp.broadcast_to(nrm, (8, num_nodes))

    return pl.pallas_call(
        col_body,
        grid=(1,),
        in_specs=[
            pl.BlockSpec(memory_space=pl.ANY),
            pl.BlockSpec((num_workers, num_nodes), lambda i: (0, 0)),
        ],
        out_specs=pl.BlockSpec((8, num_nodes), lambda i: (d // 8, 0)),
        out_shape=jax.ShapeDtypeStruct((d + 1, num_nodes), jnp.float32),
        input_output_aliases={0: 0},
    )(staged, partials)


def kernel(x, edge_index):
    num_nodes = x.shape[0]
    num_workers = 32

    edges = edge_index
    if edges.dtype != jnp.int32:
        edges = edges.astype(jnp.int32)
    partials = _sc_partial_hist(edges, num_nodes, num_workers)
    out_t = _tc_finish(x, partials, block_rows=1280)
    return jnp.transpose(out_t, (1, 0))


# speedup vs baseline: 1.0750x; 1.0608x over previous
import functools

import jax
import jax.numpy as jnp
from jax import lax
from jax.experimental import pallas as pl
from jax.experimental.pallas import tpu as pltpu
from jax.experimental.pallas import tpu_sc as plsc

_LANES = 16


def _sc_partial_hist(edges, num_nodes, num_workers):
    num_edges = edges.shape[1]
    lane_tiles = num_edges // 128
    tiles_main = lane_tiles // num_workers
    chunk_main = tiles_main * 128
    chunk_last = num_edges - chunk_main * (num_workers - 1)
    assert chunk_main % _LANES == 0 and chunk_last % _LANES == 0

    mesh = plsc.VectorSubcoreMesh(core_axis_name="c", subcore_axis_name="s")
    num_cores = mesh.num_cores

    @functools.partial(
        pl.kernel,
        out_type=jax.ShapeDtypeStruct((num_workers, num_nodes), jnp.float32),
        mesh=mesh,
        scratch_types=[
            pltpu.VMEM((2, chunk_last), jnp.int32),
            pltpu.VMEM((num_nodes,), jnp.float32),
            pltpu.SemaphoreType.DMA,
        ],
        compiler_params=pltpu.CompilerParams(needs_layout_passes=False),
    )
    def hist_kernel(edge_hbm, out_hbm, idx_v, acc_v, sem):
        wid = lax.axis_index("s") * num_cores + lax.axis_index("c")
        base = pl.multiple_of(wid * chunk_main, 128)
        cp = pltpu.async_copy(edge_hbm.at[:, pl.ds(base, chunk_last)], idx_v, sem)

        zeros = jnp.zeros((_LANES,), jnp.float32)

        @plsc.parallel_loop(0, num_nodes // _LANES, unroll=8)
        def _zero(i):
            acc_v[pl.ds(i * _LANES, _LANES)] = zeros

        cp.wait()

        ones = jnp.ones((_LANES,), jnp.float32)

        def scat_body(i):
            idx = idx_v[0, pl.ds(i * _LANES, _LANES)]
            plsc.addupdate_scatter(acc_v, [idx], ones)

        plsc.parallel_loop(0, chunk_main // _LANES, unroll=8)(scat_body)

        @pl.when(wid == num_workers - 1)
        def _():
            plsc.parallel_loop(
                chunk_main // _LANES, chunk_last // _LANES, unroll=8
            )(scat_body)

        pltpu.sync_copy(acc_v, out_hbm.at[wid])

    return hist_kernel(edges)


def _tc_finish(x, partials, block_rows):
    num_nodes, d = x.shape
    num_workers = partials.shape[0]
    num_blocks = pl.cdiv(num_nodes, block_rows)

    def copy_body(x_ref, out_ref):
        out_ref[...] = jnp.transpose(x_ref[...], (1, 0))

    staged = pl.pallas_call(
        copy_body,
        grid=(num_blocks,),
        in_specs=[pl.BlockSpec((block_rows, d), lambda i: (i, 0))],
        out_specs=pl.BlockSpec((d, block_rows), lambda i: (0, i)),
        out_shape=jax.ShapeDtypeStruct((d + 1, num_nodes), jnp.float32),
    )(x)

    assert d % 8 == 0
    def col_body(staged_ref, part_ref, out_ref):
        del staged_ref
        deg = jnp.sum(part_ref[...], axis=0, keepdims=True)
        m = jnp.max(deg)
        nrm = jnp.where(m > 0, deg / m, deg)
        out_ref[...] = jnp.broadcast_to(nrm, (8, num_nodes))

    return pl.pallas_call(
        col_body,
        grid=(1,),
        in_specs=[
            pl.BlockSpec(memory_space=pl.ANY),
            pl.BlockSpec((num_workers, num_nodes), lambda i: (0, 0)),
        ],
        out_specs=pl.BlockSpec((8, num_nodes), lambda i: (d // 8, 0)),
        out_shape=jax.ShapeDtypeStruct((d + 1, num_nodes), jnp.float32),
        input_output_aliases={0: 0},
    )(staged, partials)


def kernel(x, edge_index):
    num_nodes = x.shape[0]
    num_workers = 32

    edges = edge_index
    if edges.dtype != jnp.int32:
        edges = edges.astype(jnp.int32)
    partials = _sc_partial_hist(edges, num_nodes, num_workers)
    out_t = _tc_finish(x, partials, block_rows=1280)
    return jnp.transpose(out_t, (1, 0))
